# baseline (speedup 1.0000x reference)
import jax
import jax.numpy as jnp
from jax import lax
from jax.experimental import pallas as pl
from jax.experimental.pallas import tpu as pltpu
from jax.experimental.pallas import tpu_sc as plsc

EMBED_DIM = 1024
NC = 2
NS = 16
NW = NC * NS

CHUNK = 16
NBUF = 4
LAG = 2

TC_ROWS = 4096
TC_BLOCK = 256


def _make_sc_gather(b_total):
    b_per_w = b_total // NW
    nchunk = b_per_w // CHUNK
    n_main = nchunk - NBUF
    assert n_main % NBUF == 0 and LAG < NBUF

    mesh = plsc.VectorSubcoreMesh(core_axis_name="c", subcore_axis_name="s")

    def body(table_hbm, idx_hbm, out_hbm, idx_v, rows_v, *sems):
        sem_in = sems[:NBUF]
        sem_out = sems[NBUF:]
        wid = lax.axis_index("s") * NC + lax.axis_index("c")
        base = wid * b_per_w

        pltpu.sync_copy(idx_hbm.at[wid], idx_v)

        def start_in(g, b):
            pltpu.async_copy(table_hbm.at[idx_v.at[g]], rows_v.at[b], sem_in[b])

        def wait_in(g, b):
            pltpu.make_async_copy(
                table_hbm.at[idx_v.at[g]], rows_v.at[b], sem_in[b]).wait()

        def start_out(g, b):
            pltpu.async_copy(
                rows_v.at[b], out_hbm.at[pl.ds(base + g * CHUNK, CHUNK)],
                sem_out[b])

        def wait_out(g, b):
            pltpu.make_async_copy(
                rows_v.at[b], out_hbm.at[pl.ds(base + g * CHUNK, CHUNK)],
                sem_out[b]).wait()

        for b in range(NBUF):
            start_in(b, b)
        for g in range(LAG):
            wait_in(g, g % NBUF)
            start_out(g, g % NBUF)

        def step(t, carry):
            for b in range(NBUF):
                g = LAG + t * NBUF + b
                bg = (LAG + b) % NBUF
                wait_in(g, bg)
                start_out(g, bg)
                wait_out(g - LAG, b)
                start_in(g - LAG + NBUF, b)
            return carry

        lax.fori_loop(0, n_main // NBUF, step, 0, unroll=False)

        for g in range(nchunk - NBUF + LAG, nchunk):
            wait_in(g, g % NBUF)
            start_out(g, g % NBUF)
        for g in range(nchunk - NBUF, nchunk):
            wait_out(g, g % NBUF)

    scratch = [
        pltpu.VMEM((nchunk, CHUNK), jnp.int32),
        pltpu.VMEM((NBUF, CHUNK, EMBED_DIM), jnp.float32),
    ] + [pltpu.SemaphoreType.DMA] * (2 * NBUF)

    return pl.kernel(
        body,
        out_type=jax.ShapeDtypeStruct((b_total, EMBED_DIM), jnp.float32),
        mesh=mesh,
        scratch_types=scratch,
    )


def _tc_body(idx_smem, table_ref, out_ref):
    i = pl.program_id(0)

    def row(r, carry):
        idx = idx_smem[i * TC_BLOCK + r]
        out_ref[pl.ds(r, 1)] = table_ref[pl.ds(idx, 1)]
        return carry

    lax.fori_loop(0, TC_BLOCK, row, 0, unroll=8)


def _tc_gather(idx_tail, table3):
    grid_spec = pltpu.PrefetchScalarGridSpec(
        num_scalar_prefetch=1,
        grid=(TC_ROWS // TC_BLOCK,),
        in_specs=[
            pl.BlockSpec(table3.shape, lambda i, idx_ref: (0, 0, 0)),
        ],
        out_specs=pl.BlockSpec(
            (TC_BLOCK, 8, 128), lambda i, idx_ref: (i, 0, 0)),
    )
    return pl.pallas_call(
        _tc_body,
        grid_spec=grid_spec,
        out_shape=jax.ShapeDtypeStruct((TC_ROWS, 8, 128), jnp.float32),
    )(idx_tail, table3)


@jax.jit
def kernel(position_ids, table):
    b_total = position_ids.size
    flat = position_ids.reshape(b_total).astype(jnp.int32)
    s = b_total - TC_ROWS
    idx_head = flat[:s].reshape(NW, (s // NW) // CHUNK, CHUNK)
    sc_out = _make_sc_gather(s)(table, idx_head)
    tc_out = _tc_gather(flat[s:], table.reshape(8192, 8, 128))
    out = jnp.concatenate([sc_out, tc_out.reshape(TC_ROWS, EMBED_DIM)], axis=0)
    return out.reshape(position_ids.shape + (EMBED_DIM,))


# speedup vs baseline: 2.2123x; 2.2123x over previous
import jax
import jax.numpy as jnp
from jax import lax
from jax.experimental import pallas as pl
from jax.experimental.pallas import tpu as pltpu
from jax.experimental.pallas import tpu_sc as plsc

EMBED_DIM = 1024
NC = 2
NS = 16
NW = NC * NS

CHUNK = 32
NBUF = 2
LAG = 1


def _make_gather(b_total):
    b_per_w = b_total // NW
    nchunk = b_per_w // CHUNK
    n_main = nchunk - NBUF
    assert n_main % NBUF == 0 and LAG < NBUF

    mesh = plsc.VectorSubcoreMesh(core_axis_name="c", subcore_axis_name="s")

    def body(table_hbm, idx_hbm, out_hbm, idx_v, rows_v, *sems):
        sem_in = sems[:NBUF]
        sem_out = sems[NBUF:]
        wid = lax.axis_index("s") * NC + lax.axis_index("c")
        base = wid * b_per_w

        pltpu.sync_copy(idx_hbm.at[wid], idx_v)

        def start_in(g, b):
            pltpu.async_copy(table_hbm.at[idx_v.at[g]], rows_v.at[b], sem_in[b])

        def wait_in(g, b):
            pltpu.make_async_copy(
                table_hbm.at[idx_v.at[g]], rows_v.at[b], sem_in[b]).wait()

        def start_out(g, b):
            pltpu.async_copy(
                rows_v.at[b], out_hbm.at[pl.ds(base + g * CHUNK, CHUNK)],
                sem_out[b])

        def wait_out(g, b):
            pltpu.make_async_copy(
                rows_v.at[b], out_hbm.at[pl.ds(base + g * CHUNK, CHUNK)],
                sem_out[b]).wait()

        for b in range(NBUF):
            start_in(b, b)
        for g in range(LAG):
            wait_in(g, g % NBUF)
            start_out(g, g % NBUF)

        def step(t, carry):
            for b in range(NBUF):
                g = LAG + t * NBUF + b
                bg = (LAG + b) % NBUF
                wait_in(g, bg)
                start_out(g, bg)
                wait_out(g - LAG, b)
                start_in(g - LAG + NBUF, b)
            return carry

        lax.fori_loop(0, n_main // NBUF, step, 0, unroll=False)

        for g in range(nchunk - NBUF + LAG, nchunk):
            wait_in(g, g % NBUF)
            start_out(g, g % NBUF)
        for g in range(nchunk - NBUF, nchunk):
            wait_out(g, g % NBUF)

    scratch = [
        pltpu.VMEM((nchunk, CHUNK), jnp.int32),
        pltpu.VMEM((NBUF, CHUNK, EMBED_DIM), jnp.float32),
    ] + [pltpu.SemaphoreType.DMA] * (2 * NBUF)

    return pl.kernel(
        body,
        out_type=jax.ShapeDtypeStruct((b_total, EMBED_DIM), jnp.float32),
        mesh=mesh,
        scratch_types=scratch,
    )


@jax.jit
def kernel(position_ids, table):
    b_total = position_ids.size
    idx = position_ids.reshape(NW, (b_total // NW) // CHUNK, CHUNK)
    idx = idx.astype(jnp.int32)
    out = _make_gather(b_total)(table, idx)
    return out.reshape(position_ids.shape + (EMBED_DIM,))


# speedup vs baseline: 2.2683x; 1.0253x over previous
import jax
import jax.numpy as jnp
from jax import lax
from jax.experimental import pallas as pl
from jax.experimental.pallas import tpu as pltpu
from jax.experimental.pallas import tpu_sc as plsc

EMBED_DIM = 1024
NC = 2
NS = 16
NW = NC * NS

CHUNK = 8
NBUF = 8
LAG = 3


def _make_gather(b_total):
    b_per_w = b_total // NW
    nchunk = b_per_w // CHUNK
    n_main = nchunk - NBUF
    assert n_main % NBUF == 0 and LAG < NBUF

    mesh = plsc.VectorSubcoreMesh(core_axis_name="c", subcore_axis_name="s")

    def body(table_hbm, idx_hbm, out_hbm, idx_v, rows_v, *sems):
        sem_in = sems[:NBUF]
        sem_out = sems[NBUF:]
        wid = lax.axis_index("s") * NC + lax.axis_index("c")
        base = wid * b_per_w

        pltpu.sync_copy(idx_hbm.at[wid], idx_v)

        def start_in(g, b):
            pltpu.async_copy(table_hbm.at[idx_v.at[g]], rows_v.at[b], sem_in[b])

        def wait_in(g, b):
            pltpu.make_async_copy(
                table_hbm.at[idx_v.at[g]], rows_v.at[b], sem_in[b]).wait()

        def start_out(g, b):
            pltpu.async_copy(
                rows_v.at[b], out_hbm.at[pl.ds(base + g * CHUNK, CHUNK)],
                sem_out[b])

        def wait_out(g, b):
            pltpu.make_async_copy(
                rows_v.at[b], out_hbm.at[pl.ds(base + g * CHUNK, CHUNK)],
                sem_out[b]).wait()

        for b in range(NBUF):
            start_in(b, b)
        for g in range(LAG):
            wait_in(g, g % NBUF)
            start_out(g, g % NBUF)

        def step(t, carry):
            for b in range(NBUF):
                g = LAG + t * NBUF + b
                bg = (LAG + b) % NBUF
                wait_in(g, bg)
                start_out(g, bg)
                wait_out(g - LAG, b)
                start_in(g - LAG + NBUF, b)
            return carry

        lax.fori_loop(0, n_main // NBUF, step, 0, unroll=False)

        for g in range(nchunk - NBUF + LAG, nchunk):
            wait_in(g, g % NBUF)
            start_out(g, g % NBUF)
        for g in range(nchunk - NBUF, nchunk):
            wait_out(g, g % NBUF)

    scratch = [
        pltpu.VMEM((nchunk, CHUNK), jnp.int32),
        pltpu.VMEM((NBUF, CHUNK, EMBED_DIM), jnp.float32),
    ] + [pltpu.SemaphoreType.DMA] * (2 * NBUF)

    return pl.kernel(
        body,
        out_type=jax.ShapeDtypeStruct((b_total, EMBED_DIM), jnp.float32),
        mesh=mesh,
        scratch_types=scratch,
    )


@jax.jit
def kernel(position_ids, table):
    b_total = position_ids.size
    idx = position_ids.reshape(NW, (b_total // NW) // CHUNK, CHUNK)
    idx = idx.astype(jnp.int32)
    out = _make_gather(b_total)(table, idx)
    return out.reshape(position_ids.shape + (EMBED_DIM,))
